# baseline (speedup 1.0000x reference)
import functools

import jax
import jax.numpy as jnp
from jax import lax
from jax.experimental import pallas as pl
from jax.experimental.pallas import tpu as pltpu
from jax.experimental.pallas import tpu_sc as plsc

N = 2097152
NC = 2
NS = 16
NW = NC * NS
CHUNK = N // NW
LANES = 16
_U = 8
HALF = CHUNK // 2
SUB = CHUNK // 4
SUB_GROUPS = SUB // (_U * LANES)

_mesh = plsc.VectorSubcoreMesh(core_axis_name="c", subcore_axis_name="s")
_params = pltpu.CompilerParams(needs_layout_passes=False)


def _wid():
    return lax.axis_index("c") * NS + lax.axis_index("s")


def _tc_sums_body(x_ref, o_ref):
    i = pl.program_id(0)

    @pl.when(i == 0)
    def _():
        o_ref[...] = jnp.zeros_like(o_ref)

    s = jnp.sum(x_ref[...])
    o_ref[...] += jnp.where(lax.iota(jnp.int32, NW) == i, s, 0.0)


_chunk_sums_tc = pl.pallas_call(
    _tc_sums_body,
    grid=(NW,),
    in_specs=[pl.BlockSpec((CHUNK,), lambda i: (i,))],
    out_specs=pl.BlockSpec((NW,), lambda i: (0,)),
    out_shape=jax.ShapeDtypeStruct((NW,), jnp.float32),
)


@functools.partial(
    pl.kernel,
    out_type=jax.ShapeDtypeStruct((N,), jnp.float32),
    mesh=_mesh,
    compiler_params=_params,
    scratch_types=[
        pltpu.VMEM((SUB,), jnp.float32),
        pltpu.VMEM((SUB,), jnp.float32),
        pltpu.VMEM((SUB,), jnp.float32),
        pltpu.VMEM((SUB,), jnp.float32),
        pltpu.VMEM((NW,), jnp.float32),
        pltpu.SemaphoreType.DMA,
        pltpu.SemaphoreType.DMA,
        pltpu.SemaphoreType.DMA,
        pltpu.SemaphoreType.DMA,
    ],
)
def _scan_chunks(x_hbm, sums_hbm, out_hbm, in0, in1, out0, out1, sums_v,
                 isem0, isem1, osem0, osem1):
    wid = _wid()
    base = wid * CHUNK
    ins = (in0, in1)
    outs = (out0, out1)
    isems = (isem0, isem1)
    osems = (osem0, osem1)

    in_copies = [None] * 4
    out_copies = [None] * 4
    for b in range(2):
        in_copies[b] = pltpu.async_copy(
            x_hbm.at[pl.ds(base + b * SUB, SUB)], ins[b], isems[b])

    pltpu.sync_copy(sums_hbm, sums_v)

    lane = lax.iota(jnp.int32, LANES)
    zv = jnp.zeros((LANES,), jnp.float32)
    v0 = jnp.where(lane < wid, sums_v[pl.ds(0, LANES)], zv)
    v1 = jnp.where(lane + LANES < wid, sums_v[pl.ds(LANES, LANES)], zv)
    carry = jnp.sum(v0 + v1) - 1.0

    for b in range(4):
        in_copies[b].wait()
        if b >= 2:
            out_copies[b - 2].wait()
        ibuf = ins[b % 2]
        obuf = outs[b % 2]

        @plsc.parallel_loop(0, SUB_GROUPS, carry=carry)
        def body(g, c):
            o = g * (_U * LANES)
            ss = []
            ts = []
            for j in range(_U):
                v = ibuf[pl.ds(o + j * LANES, LANES)]
                s = jnp.cumsum(v)
                ss.append(s)
                ts.append(s[15])
            t01 = ts[0] + ts[1]
            t23 = ts[2] + ts[3]
            t45 = ts[4] + ts[5]
            t67 = ts[6] + ts[7]
            t03 = t01 + t23
            e = [None] * _U
            e[1] = ts[0]
            e[2] = t01
            e[3] = t01 + ts[2]
            e[4] = t03
            e[5] = t03 + ts[4]
            e[6] = t03 + t45
            e[7] = e[6] + ts[6]
            obuf[pl.ds(o, LANES)] = ss[0] + c
            for j in range(1, _U):
                obuf[pl.ds(o + j * LANES, LANES)] = ss[j] + (c + e[j])
            return c + (t03 + (t45 + t67))

        carry = body
        out_copies[b] = pltpu.async_copy(
            obuf, out_hbm.at[pl.ds(base + b * SUB, SUB)], osems[b % 2])
        if b + 2 < 4:
            in_copies[b + 2] = pltpu.async_copy(
                x_hbm.at[pl.ds(base + (b + 2) * SUB, SUB)], ins[b % 2], isems[b % 2])

    out_copies[2].wait()
    out_copies[3].wait()


def kernel(mask_i):
    sums = _chunk_sums_tc(mask_i)
    return _scan_chunks(mask_i, sums)


# speedup vs baseline: 1.3542x; 1.3542x over previous
import functools

import jax
import jax.numpy as jnp
from jax import lax
from jax.experimental import pallas as pl
from jax.experimental.pallas import tpu as pltpu
from jax.experimental.pallas import tpu_sc as plsc

N = 2097152
NC = 2
NS = 16
NW = NC * NS
CHUNK = N // NW
LANES = 16
_U = 8
HALF = CHUNK // 2
SUB = CHUNK // 4
SUB_GROUPS = SUB // (_U * LANES)

_mesh = plsc.VectorSubcoreMesh(core_axis_name="c", subcore_axis_name="s")
_params = pltpu.CompilerParams(needs_layout_passes=False)


def _wid():
    return lax.axis_index("c") * NS + lax.axis_index("s")


@functools.partial(
    pl.kernel,
    out_type=jax.ShapeDtypeStruct((NW * LANES,), jnp.float32),
    mesh=_mesh,
    compiler_params=_params,
    scratch_types=[
        pltpu.VMEM((HALF,), jnp.float32),
        pltpu.VMEM((HALF,), jnp.float32),
        pltpu.VMEM((LANES,), jnp.float32),
        pltpu.SemaphoreType.DMA,
        pltpu.SemaphoreType.DMA,
    ],
)
def _chunk_sums(x_hbm, out_hbm, buf0, buf1, accv, sem0, sem1):
    wid = _wid()
    base = wid * CHUNK
    bufs = (buf0, buf1)
    sems = (sem0, sem1)
    copies = [
        pltpu.async_copy(x_hbm.at[pl.ds(base + h * HALF, HALF)], bufs[h], sems[h])
        for h in range(2)
    ]

    z = jnp.zeros((LANES,), jnp.float32)
    accs = (z, z, z, z)
    for h in range(2):
        copies[h].wait()
        buf = bufs[h]

        @plsc.parallel_loop(0, HALF // (_U * LANES), carry=accs)
        def hbody(g, a):
            a0, a1, a2, a3 = a
            o = g * (_U * LANES)
            a0 = a0 + buf[pl.ds(o + 0 * LANES, LANES)]
            a1 = a1 + buf[pl.ds(o + 1 * LANES, LANES)]
            a2 = a2 + buf[pl.ds(o + 2 * LANES, LANES)]
            a3 = a3 + buf[pl.ds(o + 3 * LANES, LANES)]
            a0 = a0 + buf[pl.ds(o + 4 * LANES, LANES)]
            a1 = a1 + buf[pl.ds(o + 5 * LANES, LANES)]
            a2 = a2 + buf[pl.ds(o + 6 * LANES, LANES)]
            a3 = a3 + buf[pl.ds(o + 7 * LANES, LANES)]
            return (a0, a1, a2, a3)

        accs = hbody

    a0, a1, a2, a3 = accs
    accv[...] = (a0 + a1) + (a2 + a3)
    pltpu.sync_copy(accv, out_hbm.at[pl.ds(wid * LANES, LANES)])


@functools.partial(
    pl.kernel,
    out_type=jax.ShapeDtypeStruct((N,), jnp.float32),
    mesh=_mesh,
    compiler_params=_params,
    scratch_types=[
        pltpu.VMEM((SUB,), jnp.float32),
        pltpu.VMEM((SUB,), jnp.float32),
        pltpu.VMEM((SUB,), jnp.float32),
        pltpu.VMEM((SUB,), jnp.float32),
        pltpu.VMEM((NW * LANES,), jnp.float32),
        pltpu.SemaphoreType.DMA,
        pltpu.SemaphoreType.DMA,
        pltpu.SemaphoreType.DMA,
        pltpu.SemaphoreType.DMA,
    ],
)
def _scan_chunks(x_hbm, sums_hbm, out_hbm, in0, in1, out0, out1, sums_v,
                 isem0, isem1, osem0, osem1):
    wid = _wid()
    base = wid * CHUNK
    ins = (in0, in1)
    outs = (out0, out1)
    isems = (isem0, isem1)
    osems = (osem0, osem1)

    in_copies = [None] * 4
    out_copies = [None] * 4
    for b in range(2):
        in_copies[b] = pltpu.async_copy(
            x_hbm.at[pl.ds(base + b * SUB, SUB)], ins[b], isems[b])

    pltpu.sync_copy(sums_hbm, sums_v)

    def off_body(w, acc):
        v = sums_v[pl.ds(w * LANES, LANES)]
        keep = (w < wid).astype(jnp.float32)
        return acc + v * keep

    offv = lax.fori_loop(0, NW, off_body, jnp.zeros((LANES,), jnp.float32))
    carry = jnp.sum(offv) - 1.0

    for b in range(4):
        in_copies[b].wait()
        if b >= 2:
            out_copies[b - 2].wait()
        ibuf = ins[b % 2]
        obuf = outs[b % 2]

        @plsc.parallel_loop(0, SUB_GROUPS, unroll=2, carry=carry)
        def body(g, c):
            o = g * (_U * LANES)
            ss = []
            ts = []
            for j in range(_U):
                v = ibuf[pl.ds(o + j * LANES, LANES)]
                s = jnp.cumsum(v)
                ss.append(s)
                ts.append(s[15])
            t01 = ts[0] + ts[1]
            t23 = ts[2] + ts[3]
            t45 = ts[4] + ts[5]
            t67 = ts[6] + ts[7]
            t03 = t01 + t23
            e = [None] * _U
            e[1] = ts[0]
            e[2] = t01
            e[3] = t01 + ts[2]
            e[4] = t03
            e[5] = t03 + ts[4]
            e[6] = t03 + t45
            e[7] = e[6] + ts[6]
            obuf[pl.ds(o, LANES)] = ss[0] + c
            for j in range(1, _U):
                obuf[pl.ds(o + j * LANES, LANES)] = ss[j] + (c + e[j])
            return c + (t03 + (t45 + t67))

        carry = body
        out_copies[b] = pltpu.async_copy(
            obuf, out_hbm.at[pl.ds(base + b * SUB, SUB)], osems[b % 2])
        if b + 2 < 4:
            in_copies[b + 2] = pltpu.async_copy(
                x_hbm.at[pl.ds(base + (b + 2) * SUB, SUB)], ins[b % 2], isems[b % 2])

    out_copies[2].wait()
    out_copies[3].wait()


def kernel(mask_i):
    sums = _chunk_sums(mask_i)
    return _scan_chunks(mask_i, sums)


# speedup vs baseline: 1.3613x; 1.0052x over previous
import functools

import jax
import jax.numpy as jnp
from jax import lax
from jax.experimental import pallas as pl
from jax.experimental.pallas import tpu as pltpu
from jax.experimental.pallas import tpu_sc as plsc

N = 2097152
NC = 2
NS = 16
NW = NC * NS
CHUNK = N // NW
LANES = 16
HALF = CHUNK // 2
SUB = CHUNK // 4
SS = 256
NG = 4
NREG = CHUNK // SS
REG_H = HALF // SS

_mesh = plsc.VectorSubcoreMesh(core_axis_name="c", subcore_axis_name="s")
_params = pltpu.CompilerParams(needs_layout_passes=False)


def _wid():
    return lax.axis_index("c") * NS + lax.axis_index("s")


@functools.partial(
    pl.kernel,
    out_type=(
        jax.ShapeDtypeStruct((NW * LANES,), jnp.float32),
        jax.ShapeDtypeStruct((NW * NREG,), jnp.float32),
    ),
    mesh=_mesh,
    compiler_params=_params,
    scratch_types=[
        pltpu.VMEM((HALF,), jnp.float32),
        pltpu.VMEM((HALF,), jnp.float32),
        pltpu.VMEM((LANES,), jnp.float32),
        pltpu.VMEM((NREG,), jnp.float32),
        pltpu.SemaphoreType.DMA,
        pltpu.SemaphoreType.DMA,
    ],
)
def _chunk_sums(x_hbm, out_hbm, tot_hbm, buf0, buf1, accv, totv, sem0, sem1):
    wid = _wid()
    base = wid * CHUNK
    bufs = (buf0, buf1)
    sems = (sem0, sem1)
    copies = [
        pltpu.async_copy(x_hbm.at[pl.ds(base + h * HALF, HALF)], bufs[h], sems[h])
        for h in range(2)
    ]

    lane = lax.iota(jnp.int32, LANES)
    mask0 = lane == 0
    acc_glob = jnp.zeros((LANES,), jnp.float32)
    for h in range(2):
        copies[h].wait()
        buf = bufs[h]

        @plsc.parallel_loop(0, REG_H, carry=acc_glob)
        def rbody(j, ag):
            o = j * SS
            a0 = buf[pl.ds(o + 0 * LANES, LANES)]
            a1 = buf[pl.ds(o + 1 * LANES, LANES)]
            a2 = buf[pl.ds(o + 2 * LANES, LANES)]
            a3 = buf[pl.ds(o + 3 * LANES, LANES)]
            for t in range(4, SS // LANES):
                if t % 4 == 0:
                    a0 = a0 + buf[pl.ds(o + t * LANES, LANES)]
                elif t % 4 == 1:
                    a1 = a1 + buf[pl.ds(o + t * LANES, LANES)]
                elif t % 4 == 2:
                    a2 = a2 + buf[pl.ds(o + t * LANES, LANES)]
                else:
                    a3 = a3 + buf[pl.ds(o + t * LANES, LANES)]
            ra = (a0 + a1) + (a2 + a3)
            t = jnp.sum(ra)
            plsc.store_scatter(
                totv,
                [jnp.full((LANES,), h * REG_H + j, jnp.int32)],
                jnp.full((LANES,), t),
                mask=mask0,
            )
            return ag + ra

        acc_glob = rbody

    accv[...] = acc_glob
    pltpu.sync_copy(accv, out_hbm.at[pl.ds(wid * LANES, LANES)])
    pltpu.sync_copy(totv, tot_hbm.at[pl.ds(wid * NREG, NREG)])


@functools.partial(
    pl.kernel,
    out_type=jax.ShapeDtypeStruct((N,), jnp.float32),
    mesh=_mesh,
    compiler_params=_params,
    scratch_types=[
        pltpu.VMEM((SUB,), jnp.float32),
        pltpu.VMEM((SUB,), jnp.float32),
        pltpu.VMEM((SUB,), jnp.float32),
        pltpu.VMEM((SUB,), jnp.float32),
        pltpu.VMEM((NW * LANES,), jnp.float32),
        pltpu.VMEM((NREG,), jnp.float32),
        pltpu.SemaphoreType.DMA,
        pltpu.SemaphoreType.DMA,
        pltpu.SemaphoreType.DMA,
        pltpu.SemaphoreType.DMA,
    ],
)
def _scan_chunks(x_hbm, sums_hbm, tots_hbm, out_hbm, in0, in1, out0, out1,
                 sums_v, tot_v, isem0, isem1, osem0, osem1):
    wid = _wid()
    base = wid * CHUNK
    ins = (in0, in1)
    outs = (out0, out1)
    isems = (isem0, isem1)
    osems = (osem0, osem1)

    in_copies = [None] * 4
    out_copies = [None] * 4
    for b in range(2):
        in_copies[b] = pltpu.async_copy(
            x_hbm.at[pl.ds(base + b * SUB, SUB)], ins[b], isems[b])

    pltpu.sync_copy(sums_hbm, sums_v)
    pltpu.sync_copy(tots_hbm.at[pl.ds(wid * NREG, NREG)], tot_v)

    def off_body(w, acc):
        v = sums_v[pl.ds(w * LANES, LANES)]
        keep = (w < wid).astype(jnp.float32)
        return acc + v * keep

    offv = lax.fori_loop(0, NW, off_body, jnp.zeros((LANES,), jnp.float32))
    carry = jnp.sum(offv) - 1.0

    lane = lax.iota(jnp.int32, LANES)

    for b in range(4):
        in_copies[b].wait()
        if b >= 2:
            out_copies[b - 2].wait()
        ibuf = ins[b % 2]
        obuf = outs[b % 2]

        rt = [tot_v[pl.ds(b * NG * LANES + q * LANES, LANES)] for q in range(NG)]
        rs = [plsc.cumsum(r) for r in rt]
        bases = [carry]
        for q in range(1, NG):
            bases.append(bases[q - 1] + rs[q - 1][15])
        runs = tuple((rs[q] - rt[q]) + bases[q] for q in range(NG))
        carry = bases[NG - 1] + rs[NG - 1][15]
        ibase = [(lane + q * LANES) * SS - lane for q in range(NG)]

        def edge(k, rc, m):
            new = []
            for q in range(NG):
                idx = ibase[q] + k
                v = plsc.load_gather(ibuf, [idx], mask=m)
                r2 = rc[q] + jnp.where(m, v, 0.0)
                plsc.store_scatter(obuf, [idx], r2, mask=m)
                new.append(r2)
            return tuple(new)

        runs = lax.fori_loop(
            0, LANES - 1, lambda k, rc: edge(k, rc, lane <= k), runs)

        @plsc.parallel_loop(LANES - 1, SS, carry=runs)
        def mbody(k, rc):
            new = []
            for q in range(NG):
                idx = ibase[q] + k
                v = plsc.load_gather(ibuf, [idx])
                r2 = rc[q] + v
                plsc.store_scatter(obuf, [idx], r2)
                new.append(r2)
            return tuple(new)

        runs = lax.fori_loop(
            SS, SS + LANES - 1, lambda k, rc: edge(k, rc, lane > k - SS), mbody)

        out_copies[b] = pltpu.async_copy(
            obuf, out_hbm.at[pl.ds(base + b * SUB, SUB)], osems[b % 2])
        if b + 2 < 4:
            in_copies[b + 2] = pltpu.async_copy(
                x_hbm.at[pl.ds(base + (b + 2) * SUB, SUB)], ins[b % 2], isems[b % 2])

    out_copies[2].wait()
    out_copies[3].wait()


def kernel(mask_i):
    sums, tots = _chunk_sums(mask_i)
    return _scan_chunks(mask_i, sums, tots)
